# Initial kernel scaffold; baseline (speedup 1.0000x reference)
import jax
import jax.numpy as jnp
from jax.experimental import pallas as pl


def kernel(input_tensor, pos_table, W, b, gamma, beta):
    raise NotImplementedError("write your pallas kernel here")


# speedup vs baseline: 3.0051x; 3.0051x over previous
import functools

import jax
import jax.numpy as jnp
from jax.experimental import pallas as pl

_EPS = 1e-12


def _fused_kernel(x_ref, pos_ref, w_ref, b_ref, g_ref, beta_ref, o_ref):
    x = x_ref[0] + pos_ref[...]
    y = jnp.dot(x, w_ref[...], preferred_element_type=jnp.float32)
    y = y + b_ref[...]
    u = jnp.mean(y, axis=-1, keepdims=True)
    d = y - u
    s = jnp.mean(d * d, axis=-1, keepdims=True)
    o_ref[0] = g_ref[...] * (d * jax.lax.rsqrt(s + _EPS)) + beta_ref[...]


@jax.jit
def kernel(input_tensor, pos_table, W, b, gamma, beta):
    batch, seq, d_in = input_tensor.shape
    d_hid = W.shape[1]
    bs = 1024
    n_seq = seq // bs

    pos = pos_table[:seq]
    b2 = b.reshape(1, d_hid)
    g2 = gamma.reshape(1, d_hid)
    beta2 = beta.reshape(1, d_hid)

    out = pl.pallas_call(
        _fused_kernel,
        grid=(n_seq, batch),
        in_specs=[
            pl.BlockSpec((1, bs, d_in), lambda i, j: (j, i, 0)),
            pl.BlockSpec((bs, d_in), lambda i, j: (i, 0)),
            pl.BlockSpec((d_in, d_hid), lambda i, j: (0, 0)),
            pl.BlockSpec((1, d_hid), lambda i, j: (0, 0)),
            pl.BlockSpec((1, d_hid), lambda i, j: (0, 0)),
            pl.BlockSpec((1, d_hid), lambda i, j: (0, 0)),
        ],
        out_specs=pl.BlockSpec((1, bs, d_hid), lambda i, j: (j, i, 0)),
        out_shape=jax.ShapeDtypeStruct((batch, seq, d_hid), jnp.float32),
    )(input_tensor, pos, W, b2, g2, beta2)
    return out
